# Initial kernel scaffold; baseline (speedup 1.0000x reference)
import jax
import jax.numpy as jnp
from jax.experimental import pallas as pl


def kernel(points, centroids, features):
    raise NotImplementedError("write your pallas kernel here")


# speedup vs baseline: 240.7063x; 240.7063x over previous
import functools

import jax
import jax.numpy as jnp
from jax import lax
from jax.experimental import pallas as pl
from jax.experimental.pallas import tpu as pltpu
from jax.experimental.pallas import tpu_sc as plsc

_NN = 32
_R2 = 0.2 * 0.2


def _rne_bf16(x):
    u = lax.bitcast_convert_type(x, jnp.uint32)
    u = u + jnp.uint32(0x7FFF) + ((u >> jnp.uint32(16)) & jnp.uint32(1))
    u = u & jnp.uint32(0xFFFF0000)
    return lax.bitcast_convert_type(u, jnp.float32)


def _transpose_in(features):
    B, C, N = features.shape
    TN = 512

    def body(f_ref, o_ref):
        o_ref[0] = f_ref[0].T

    return pl.pallas_call(
        body,
        grid=(B, N // TN),
        in_specs=[pl.BlockSpec((1, C, TN), lambda b, t: (b, 0, t))],
        out_specs=pl.BlockSpec((1, TN, C), lambda b, t: (b, t, 0)),
        out_shape=jax.ShapeDtypeStruct((B, N, C), jnp.float32),
    )(features)


def _transpose_out(rows, B, K):
    R, C = rows.shape
    TR = 512
    per_b = K * _NN

    def body(r_ref, o_ref):
        o_ref[0] = r_ref[...].T

    return pl.pallas_call(
        body,
        grid=(B, per_b // TR),
        in_specs=[pl.BlockSpec((TR, C), lambda b, t: (b * (per_b // TR) + t, 0))],
        out_specs=pl.BlockSpec((1, C, TR), lambda b, t: (b, 0, t)),
        out_shape=jax.ShapeDtypeStruct((B, C, per_b), jnp.float32),
    )(rows)


def _sc_query_group(ptx, pty, ptz, ctx, cty, ctz, ftf, B, N, K, C):
    mesh = plsc.VectorSubcoreMesh(core_axis_name="c", subcore_axis_name="s")
    NW = 32
    wpb = NW // B
    KW = K // wpb
    nchunk = N // 16
    rows_per_w = KW * _NN
    IDXW = 128
    nfc = rows_per_w // IDXW

    @functools.partial(
        pl.kernel,
        mesh=mesh,
        compiler_params=pltpu.CompilerParams(needs_layout_passes=False,
                                             use_tc_tiling_on_sc=False),
        out_type=[
            jax.ShapeDtypeStruct((B, 3, K, _NN), jnp.float32),
            jax.ShapeDtypeStruct((B * K * _NN, C), jnp.float32),
        ],
        scratch_types=[
            pltpu.VMEM((N,), jnp.float32),
            pltpu.VMEM((N,), jnp.float32),
            pltpu.VMEM((N,), jnp.float32),
            pltpu.VMEM((KW,), jnp.float32),
            pltpu.VMEM((KW,), jnp.float32),
            pltpu.VMEM((KW,), jnp.float32),
            pltpu.VMEM((48,), jnp.int32),
            pltpu.VMEM((nfc, IDXW), jnp.int32),
            pltpu.VMEM((KW, _NN), jnp.float32),
            pltpu.VMEM((KW, _NN), jnp.float32),
            pltpu.VMEM((KW, _NN), jnp.float32),
            pltpu.VMEM((IDXW, C), jnp.float32),
            pltpu.SemaphoreType.DMA,
        ],
    )
    def sc_kern(ptx_h, pty_h, ptz_h, ctx_h, cty_h, ctz_h, ftf_h,
                gxyz_h, grows_h,
                xv, yv, zv, cxv, cyv, czv, scanv, gidxv, gxv, gyv, gzv,
                fbv, sem):
        cid = lax.axis_index("c")
        sid = lax.axis_index("s")
        wid = sid * 2 + cid
        b = wid // wpb
        k0 = (wid % wpb) * KW

        pltpu.sync_copy(ptx_h.at[pl.ds(b * N, N)], xv)
        pltpu.sync_copy(pty_h.at[pl.ds(b * N, N)], yv)
        pltpu.sync_copy(ptz_h.at[pl.ds(b * N, N)], zv)
        pltpu.sync_copy(ctx_h.at[pl.ds(b * K + k0, KW)], cxv)
        pltpu.sync_copy(cty_h.at[pl.ds(b * K + k0, KW)], cyv)
        pltpu.sync_copy(ctz_h.at[pl.ds(b * K + k0, KW)], czv)

        lane = lax.iota(jnp.int32, 16)
        zero16 = jnp.zeros((16,), jnp.int32)

        def per_centroid(k, carry):
            kf = zero16 + k
            cx = plsc.load_gather(cxv, [kf])
            cy = plsc.load_gather(cyv, [kf])
            cz = plsc.load_gather(czv, [kf])
            cxb = _rne_bf16(cx)
            cyb = _rne_bf16(cy)
            czb = _rne_bf16(cz)
            c2 = (cx * cx + cy * cy) + cz * cz

            def cond(st):
                n, cnt = st
                return (cnt < _NN) & (n < nchunk)

            def body(st):
                n, cnt = st
                sl = pl.ds(n * 16, 16)
                x = xv[sl]
                y = yv[sl]
                z = zv[sl]
                p2 = (x * x + y * y) + z * z
                cp = cxb * _rne_bf16(x) + (cyb * _rne_bf16(y)
                                           + czb * _rne_bf16(z))
                d2 = (c2 + p2) - 2.0 * cp
                m = d2 <= _R2
                plsc.store_compressed(scanv.at[pl.ds(cnt, 16)],
                                      lane + n * 16, mask=m)
                return (n + 1, cnt + jnp.sum(m.astype(jnp.int32)))

            _, cnt = lax.while_loop(cond, body,
                                    (jnp.int32(0), jnp.int32(0)))

            cntv = zero16 + cnt
            for h in range(2):
                pos = lane + 16 * h
                posc = jnp.where(pos < cntv, pos, 0)
                vals = plsc.load_gather(scanv, [posc])
                vals = jnp.where(cntv > 0, vals, jnp.int32(N - 1))
                gxv[k, pl.ds(16 * h, 16)] = plsc.load_gather(xv, [vals]) - cx
                gyv[k, pl.ds(16 * h, 16)] = plsc.load_gather(yv, [vals]) - cy
                gzv[k, pl.ds(16 * h, 16)] = plsc.load_gather(zv, [vals]) - cz
                slot = k * _NN + 16 * h
                gidxv[slot // IDXW, pl.ds(slot % IDXW, 16)] = vals + b * N
            return carry

        lax.fori_loop(0, KW, per_centroid, 0)

        pltpu.sync_copy(gxv, gxyz_h.at[b, 0, pl.ds(k0, KW), :])
        pltpu.sync_copy(gyv, gxyz_h.at[b, 1, pl.ds(k0, KW), :])
        pltpu.sync_copy(gzv, gxyz_h.at[b, 2, pl.ds(k0, KW), :])

        row0 = (b * K + k0) * _NN

        def per_chunk(ch, carry):
            pltpu.async_copy(ftf_h.at[gidxv.at[ch]], fbv, sem).wait()
            pltpu.sync_copy(fbv, grows_h.at[pl.ds(row0 + ch * IDXW, IDXW)])
            return carry

        lax.fori_loop(0, nfc, per_chunk, 0)

    return sc_kern(ptx, pty, ptz, ctx, cty, ctz, ftf)


def kernel(points, centroids, features):
    B, N, _ = points.shape
    K = centroids.shape[1]
    C = features.shape[1]
    ptx = points[:, :, 0].reshape(B * N)
    pty = points[:, :, 1].reshape(B * N)
    ptz = points[:, :, 2].reshape(B * N)
    ctx = centroids[:, :, 0].reshape(B * K)
    cty = centroids[:, :, 1].reshape(B * K)
    ctz = centroids[:, :, 2].reshape(B * K)
    ftf = _transpose_in(features).reshape(B * N, C)
    gxyz, grows = _sc_query_group(ptx, pty, ptz, ctx, cty, ctz, ftf,
                                  B, N, K, C)
    gfeat = _transpose_out(grows, B, K).reshape(B, C, K, _NN)
    return (gxyz, gfeat)


# speedup vs baseline: 352.9080x; 1.4661x over previous
import functools

import jax
import jax.numpy as jnp
from jax import lax
from jax.experimental import pallas as pl
from jax.experimental.pallas import tpu as pltpu
from jax.experimental.pallas import tpu_sc as plsc

_NN = 32
_R2 = 0.2 * 0.2


def _rne_bf16(x):
    u = lax.bitcast_convert_type(x, jnp.uint32)
    u = u + jnp.uint32(0x7FFF) + ((u >> jnp.uint32(16)) & jnp.uint32(1))
    u = u & jnp.uint32(0xFFFF0000)
    return lax.bitcast_convert_type(u, jnp.float32)


def _transpose_in(features):
    B, C, N = features.shape
    TN = 512

    def body(f_ref, o_ref):
        o_ref[0] = f_ref[0].T

    return pl.pallas_call(
        body,
        grid=(B, N // TN),
        in_specs=[pl.BlockSpec((1, C, TN), lambda b, t: (b, 0, t))],
        out_specs=pl.BlockSpec((1, TN, C), lambda b, t: (b, t, 0)),
        out_shape=jax.ShapeDtypeStruct((B, N, C), jnp.float32),
    )(features)


def _transpose_out(rows, B, K):
    R, C = rows.shape
    TR = 512
    per_b = K * _NN

    def body(r_ref, o_ref):
        o_ref[0] = r_ref[...].T

    return pl.pallas_call(
        body,
        grid=(B, per_b // TR),
        in_specs=[pl.BlockSpec((TR, C), lambda b, t: (b * (per_b // TR) + t, 0))],
        out_specs=pl.BlockSpec((1, C, TR), lambda b, t: (b, 0, t)),
        out_shape=jax.ShapeDtypeStruct((B, C, per_b), jnp.float32),
    )(rows)


def _sc_query_group(ptx, pty, ptz, ctx, cty, ctz, ftf, B, N, K, C):
    mesh = plsc.VectorSubcoreMesh(core_axis_name="c", subcore_axis_name="s")
    NW = 32
    wpb = NW // B
    KW = K // wpb
    ncg = N // 64
    IDXW = 128
    nfc = KW * _NN // IDXW

    @functools.partial(
        pl.kernel,
        mesh=mesh,
        compiler_params=pltpu.CompilerParams(needs_layout_passes=False,
                                             use_tc_tiling_on_sc=False),
        out_type=[
            jax.ShapeDtypeStruct((B, 3, K, _NN), jnp.float32),
            jax.ShapeDtypeStruct((B * K * _NN, C), jnp.float32),
        ],
        scratch_types=[
            pltpu.VMEM((N,), jnp.float32),
            pltpu.VMEM((N,), jnp.float32),
            pltpu.VMEM((N,), jnp.float32),
            pltpu.VMEM((KW,), jnp.float32),
            pltpu.VMEM((KW,), jnp.float32),
            pltpu.VMEM((KW,), jnp.float32),
            pltpu.VMEM((128,), jnp.int32),
            pltpu.VMEM((nfc, IDXW), jnp.int32),
            pltpu.VMEM((KW, _NN), jnp.float32),
            pltpu.VMEM((KW, _NN), jnp.float32),
            pltpu.VMEM((KW, _NN), jnp.float32),
            pltpu.VMEM((2, IDXW, C), jnp.float32),
            pltpu.SemaphoreType.DMA,
            pltpu.SemaphoreType.DMA,
        ],
    )
    def sc_kern(ptx_h, pty_h, ptz_h, ctx_h, cty_h, ctz_h, ftf_h,
                gxyz_h, grows_h,
                xv, yv, zv, cxv, cyv, czv, scanv, gidxv, gxv, gyv, gzv,
                fbv, sem_g, sem_w):
        cid = lax.axis_index("c")
        sid = lax.axis_index("s")
        wid = sid * 2 + cid
        b = wid // wpb
        k0 = (wid % wpb) * KW

        pltpu.sync_copy(ptx_h.at[pl.ds(b * N, N)], xv)
        pltpu.sync_copy(pty_h.at[pl.ds(b * N, N)], yv)
        pltpu.sync_copy(ptz_h.at[pl.ds(b * N, N)], zv)
        pltpu.sync_copy(ctx_h.at[pl.ds(b * K + k0, KW)], cxv)
        pltpu.sync_copy(cty_h.at[pl.ds(b * K + k0, KW)], cyv)
        pltpu.sync_copy(ctz_h.at[pl.ds(b * K + k0, KW)], czv)

        lane = lax.iota(jnp.int32, 16)
        zero16 = jnp.zeros((16,), jnp.int32)
        row0 = (b * K + k0) * _NN

        def scan_centroid(i, j4):
            k = i * 4 + j4
            kf = zero16 + k
            cx = plsc.load_gather(cxv, [kf])
            cy = plsc.load_gather(cyv, [kf])
            cz = plsc.load_gather(czv, [kf])
            cxb = _rne_bf16(cx)
            cyb = _rne_bf16(cy)
            czb = _rne_bf16(cz)
            c2 = (cx * cx + cy * cy) + cz * cz

            def cond(st):
                g, cnt = st
                return (cnt < _NN) & (g < ncg)

            def body(st):
                g, cnt = st
                base = g * 64
                ms, sums = [], []
                for j in range(4):
                    sl = pl.ds(base + j * 16, 16)
                    x = xv[sl]
                    y = yv[sl]
                    z = zv[sl]
                    p2 = (x * x + y * y) + z * z
                    cp = cxb * _rne_bf16(x) + (cyb * _rne_bf16(y)
                                               + czb * _rne_bf16(z))
                    d2 = (c2 + p2) - 2.0 * cp
                    m = d2 <= _R2
                    ms.append(m)
                    sums.append(jnp.sum(m.astype(jnp.int32)))
                c = cnt
                for j in range(4):
                    plsc.store_compressed(scanv.at[pl.ds(c, 16)],
                                          lane + (base + j * 16), mask=ms[j])
                    c = c + sums[j]
                return (g + 1, c)

            _, cnt = lax.while_loop(cond, body,
                                    (jnp.int32(0), jnp.int32(0)))

            cntv = zero16 + cnt
            for h in range(2):
                pos = lane + 16 * h
                posc = jnp.where(pos < cntv, pos, 0)
                vals = plsc.load_gather(scanv, [posc])
                vals = jnp.where(cntv > 0, vals, jnp.int32(N - 1))
                gxv[k, pl.ds(16 * h, 16)] = plsc.load_gather(xv, [vals]) - cx
                gyv[k, pl.ds(16 * h, 16)] = plsc.load_gather(yv, [vals]) - cy
                gzv[k, pl.ds(16 * h, 16)] = plsc.load_gather(zv, [vals]) - cz
                gidxv[i, pl.ds(j4 * _NN + 16 * h, 16)] = vals + b * N

        def per_group(i, carry):
            for j4 in range(4):
                scan_centroid(i, j4)

            @pl.when(i >= 2)
            def _():
                pltpu.make_async_copy(fbv.at[0],
                                      grows_h.at[pl.ds(row0, IDXW)],
                                      sem_w).wait()

            pltpu.async_copy(ftf_h.at[gidxv.at[i]], fbv.at[i % 2], sem_g)

            @pl.when(i >= 1)
            def _():
                pltpu.make_async_copy(ftf_h.at[gidxv.at[0]], fbv.at[0],
                                      sem_g).wait()
                pltpu.async_copy(fbv.at[(i + 1) % 2],
                                 grows_h.at[pl.ds(row0 + (i - 1) * IDXW,
                                                  IDXW)],
                                 sem_w)
            return carry

        lax.fori_loop(0, nfc, per_group, 0)

        pltpu.make_async_copy(ftf_h.at[gidxv.at[0]], fbv.at[0], sem_g).wait()
        pltpu.async_copy(fbv.at[(nfc - 1) % 2],
                         grows_h.at[pl.ds(row0 + (nfc - 1) * IDXW, IDXW)],
                         sem_w)
        pltpu.make_async_copy(fbv.at[0], grows_h.at[pl.ds(row0, IDXW)],
                              sem_w).wait()
        pltpu.make_async_copy(fbv.at[0], grows_h.at[pl.ds(row0, IDXW)],
                              sem_w).wait()

        pltpu.sync_copy(gxv, gxyz_h.at[b, 0, pl.ds(k0, KW), :])
        pltpu.sync_copy(gyv, gxyz_h.at[b, 1, pl.ds(k0, KW), :])
        pltpu.sync_copy(gzv, gxyz_h.at[b, 2, pl.ds(k0, KW), :])

    return sc_kern(ptx, pty, ptz, ctx, cty, ctz, ftf)


def kernel(points, centroids, features):
    B, N, _ = points.shape
    K = centroids.shape[1]
    C = features.shape[1]
    ptx = points[:, :, 0].reshape(B * N)
    pty = points[:, :, 1].reshape(B * N)
    ptz = points[:, :, 2].reshape(B * N)
    ctx = centroids[:, :, 0].reshape(B * K)
    cty = centroids[:, :, 1].reshape(B * K)
    ctz = centroids[:, :, 2].reshape(B * K)
    ftf = _transpose_in(features).reshape(B * N, C)
    gxyz, grows = _sc_query_group(ptx, pty, ptz, ctx, cty, ctz, ftf,
                                  B, N, K, C)
    gfeat = _transpose_out(grows, B, K).reshape(B, C, K, _NN)
    return (gxyz, gfeat)


# speedup vs baseline: 494.5847x; 1.4015x over previous
import functools

import jax
import jax.numpy as jnp
from jax import lax
from jax.experimental import pallas as pl
from jax.experimental.pallas import tpu as pltpu
from jax.experimental.pallas import tpu_sc as plsc

_NN = 32
_R2 = 0.2 * 0.2


def _rne_bf16(x):
    u = lax.bitcast_convert_type(x, jnp.uint32)
    u = u + jnp.uint32(0x7FFF) + ((u >> jnp.uint32(16)) & jnp.uint32(1))
    u = u & jnp.uint32(0xFFFF0000)
    return lax.bitcast_convert_type(u, jnp.float32)


def _transpose_in(features):
    B, C, N = features.shape
    TN = 512

    def body(f_ref, o_ref):
        o_ref[0] = f_ref[0].T

    return pl.pallas_call(
        body,
        grid=(B, N // TN),
        in_specs=[pl.BlockSpec((1, C, TN), lambda b, t: (b, 0, t))],
        out_specs=pl.BlockSpec((1, TN, C), lambda b, t: (b, t, 0)),
        out_shape=jax.ShapeDtypeStruct((B, N, C), jnp.float32),
    )(features)


def _transpose_out(rows, B, K):
    R, C = rows.shape
    TK = 128
    nt = K // TK

    def body(r_ref, o_ref):
        x3 = r_ref[...].reshape(TK, _NN, C)
        for s in range(_NN):
            o_ref[0, :, s, :] = x3[:, s, :].T

    return pl.pallas_call(
        body,
        grid=(B, nt),
        in_specs=[pl.BlockSpec((TK * _NN, C), lambda b, t: (b * nt + t, 0))],
        out_specs=pl.BlockSpec((1, C, _NN, TK), lambda b, t: (b, 0, 0, t)),
        out_shape=jax.ShapeDtypeStruct((B, C, _NN, K), jnp.float32),
    )(rows)


def _sc_query_group(ptx, pty, ptz, ctx, cty, ctz, ftf, B, N, K, C):
    mesh = plsc.VectorSubcoreMesh(core_axis_name="c", subcore_axis_name="s")
    NW = 32
    wpb = NW // B
    KW = K // wpb
    ncg = N // 64
    IDXW = 128
    nfc = KW * _NN // IDXW

    @functools.partial(
        pl.kernel,
        mesh=mesh,
        compiler_params=pltpu.CompilerParams(needs_layout_passes=False,
                                             use_tc_tiling_on_sc=False),
        out_type=[
            jax.ShapeDtypeStruct((B, 3, K, _NN), jnp.float32),
            jax.ShapeDtypeStruct((B * K * _NN, C), jnp.float32),
        ],
        scratch_types=[
            pltpu.VMEM((N,), jnp.float32),
            pltpu.VMEM((N,), jnp.float32),
            pltpu.VMEM((N,), jnp.float32),
            pltpu.VMEM((KW,), jnp.float32),
            pltpu.VMEM((KW,), jnp.float32),
            pltpu.VMEM((KW,), jnp.float32),
            pltpu.VMEM((128,), jnp.int32),
            pltpu.VMEM((nfc, IDXW), jnp.int32),
            pltpu.VMEM((KW, _NN), jnp.float32),
            pltpu.VMEM((KW, _NN), jnp.float32),
            pltpu.VMEM((KW, _NN), jnp.float32),
            pltpu.VMEM((2, IDXW, C), jnp.float32),
            pltpu.SemaphoreType.DMA,
            pltpu.SemaphoreType.DMA,
        ],
    )
    def sc_kern(ptx_h, pty_h, ptz_h, ctx_h, cty_h, ctz_h, ftf_h,
                gxyz_h, grows_h,
                xv, yv, zv, cxv, cyv, czv, scanv, gidxv, gxv, gyv, gzv,
                fbv, sem_g, sem_w):
        cid = lax.axis_index("c")
        sid = lax.axis_index("s")
        wid = sid * 2 + cid
        b = wid // wpb
        k0 = (wid % wpb) * KW

        pltpu.sync_copy(ptx_h.at[pl.ds(b * N, N)], xv)
        pltpu.sync_copy(pty_h.at[pl.ds(b * N, N)], yv)
        pltpu.sync_copy(ptz_h.at[pl.ds(b * N, N)], zv)
        pltpu.sync_copy(ctx_h.at[pl.ds(b * K + k0, KW)], cxv)
        pltpu.sync_copy(cty_h.at[pl.ds(b * K + k0, KW)], cyv)
        pltpu.sync_copy(ctz_h.at[pl.ds(b * K + k0, KW)], czv)

        lane = lax.iota(jnp.int32, 16)
        zero16 = jnp.zeros((16,), jnp.int32)
        row0 = (b * K + k0) * _NN

        def scan_centroid(i, j4):
            k = i * 4 + j4
            kf = zero16 + k
            cx = plsc.load_gather(cxv, [kf])
            cy = plsc.load_gather(cyv, [kf])
            cz = plsc.load_gather(czv, [kf])
            cxb = _rne_bf16(cx)
            cyb = _rne_bf16(cy)
            czb = _rne_bf16(cz)
            c2 = (cx * cx + cy * cy) + cz * cz

            def cond(st):
                g, cnt = st
                return (cnt < _NN) & (g < ncg)

            def body(st):
                g, cnt = st
                base = g * 64
                ms, sums = [], []
                for j in range(4):
                    sl = pl.ds(base + j * 16, 16)
                    x = xv[sl]
                    y = yv[sl]
                    z = zv[sl]
                    p2 = (x * x + y * y) + z * z
                    cp = cxb * _rne_bf16(x) + (cyb * _rne_bf16(y)
                                               + czb * _rne_bf16(z))
                    d2 = (c2 + p2) - 2.0 * cp
                    m = d2 <= _R2
                    ms.append(m)
                    sums.append(jnp.sum(m.astype(jnp.int32)))
                c = cnt
                for j in range(4):
                    plsc.store_compressed(scanv.at[pl.ds(c, 16)],
                                          lane + (base + j * 16), mask=ms[j])
                    c = c + sums[j]
                return (g + 1, c)

            _, cnt = lax.while_loop(cond, body,
                                    (jnp.int32(0), jnp.int32(0)))

            cntv = zero16 + cnt
            for h in range(2):
                pos = lane + 16 * h
                posc = jnp.where(pos < cntv, pos, 0)
                vals = plsc.load_gather(scanv, [posc])
                vals = jnp.where(cntv > 0, vals, jnp.int32(N - 1))
                gxv[k, pl.ds(16 * h, 16)] = plsc.load_gather(xv, [vals]) - cx
                gyv[k, pl.ds(16 * h, 16)] = plsc.load_gather(yv, [vals]) - cy
                gzv[k, pl.ds(16 * h, 16)] = plsc.load_gather(zv, [vals]) - cz
                gidxv[i, pl.ds(j4 * _NN + 16 * h, 16)] = vals + b * N

        def per_group(i, carry):
            for j4 in range(4):
                scan_centroid(i, j4)

            @pl.when(i >= 2)
            def _():
                pltpu.make_async_copy(fbv.at[0],
                                      grows_h.at[pl.ds(row0, IDXW)],
                                      sem_w).wait()

            pltpu.async_copy(ftf_h.at[gidxv.at[i]], fbv.at[i % 2], sem_g)

            @pl.when(i >= 1)
            def _():
                pltpu.make_async_copy(ftf_h.at[gidxv.at[0]], fbv.at[0],
                                      sem_g).wait()
                pltpu.async_copy(fbv.at[(i + 1) % 2],
                                 grows_h.at[pl.ds(row0 + (i - 1) * IDXW,
                                                  IDXW)],
                                 sem_w)
            return carry

        lax.fori_loop(0, nfc, per_group, 0)

        pltpu.make_async_copy(ftf_h.at[gidxv.at[0]], fbv.at[0], sem_g).wait()
        pltpu.async_copy(fbv.at[(nfc - 1) % 2],
                         grows_h.at[pl.ds(row0 + (nfc - 1) * IDXW, IDXW)],
                         sem_w)
        pltpu.make_async_copy(fbv.at[0], grows_h.at[pl.ds(row0, IDXW)],
                              sem_w).wait()
        pltpu.make_async_copy(fbv.at[0], grows_h.at[pl.ds(row0, IDXW)],
                              sem_w).wait()

        pltpu.sync_copy(gxv, gxyz_h.at[b, 0, pl.ds(k0, KW), :])
        pltpu.sync_copy(gyv, gxyz_h.at[b, 1, pl.ds(k0, KW), :])
        pltpu.sync_copy(gzv, gxyz_h.at[b, 2, pl.ds(k0, KW), :])

    return sc_kern(ptx, pty, ptz, ctx, cty, ctz, ftf)


def kernel(points, centroids, features):
    B, N, _ = points.shape
    K = centroids.shape[1]
    C = features.shape[1]
    ptx = points[:, :, 0].reshape(B * N)
    pty = points[:, :, 1].reshape(B * N)
    ptz = points[:, :, 2].reshape(B * N)
    ctx = centroids[:, :, 0].reshape(B * K)
    cty = centroids[:, :, 1].reshape(B * K)
    ctz = centroids[:, :, 2].reshape(B * K)
    ftf = _transpose_in(features).reshape(B * N, C)
    gxyz, grows = _sc_query_group(ptx, pty, ptz, ctx, cty, ctz, ftf,
                                  B, N, K, C)
    gfeat = jnp.transpose(_transpose_out(grows, B, K), (0, 1, 3, 2))
    return (gxyz, gfeat)


# speedup vs baseline: 538.2523x; 1.0883x over previous
import functools

import jax
import jax.numpy as jnp
from jax import lax
from jax.experimental import pallas as pl
from jax.experimental.pallas import tpu as pltpu
from jax.experimental.pallas import tpu_sc as plsc

_NN = 32
_R2 = 0.2 * 0.2


def _rne_bf16(x):
    u = lax.bitcast_convert_type(x, jnp.uint32)
    u = u + jnp.uint32(0x7FFF) + ((u >> jnp.uint32(16)) & jnp.uint32(1))
    u = u & jnp.uint32(0xFFFF0000)
    return lax.bitcast_convert_type(u, jnp.float32)


def _transpose_in(features):
    B, C, N = features.shape
    TN = 512

    def body(f_ref, o_ref):
        o_ref[0] = f_ref[0].T

    return pl.pallas_call(
        body,
        grid=(B, N // TN),
        in_specs=[pl.BlockSpec((1, C, TN), lambda b, t: (b, 0, t))],
        out_specs=pl.BlockSpec((1, TN, C), lambda b, t: (b, t, 0)),
        out_shape=jax.ShapeDtypeStruct((B, N, C), jnp.float32),
    )(features)


def _transpose_out(rows, B, K):
    R, C = rows.shape
    TK = 128
    nt = K // TK

    def body(r_ref, o_ref):
        x3 = r_ref[...].reshape(TK, _NN, C)
        for s in range(_NN):
            o_ref[0, :, s, :] = x3[:, s, :].T

    return pl.pallas_call(
        body,
        grid=(B, nt),
        in_specs=[pl.BlockSpec((TK * _NN, C), lambda b, t: (b * nt + t, 0))],
        out_specs=pl.BlockSpec((1, C, _NN, TK), lambda b, t: (b, 0, 0, t)),
        out_shape=jax.ShapeDtypeStruct((B, C, _NN, K), jnp.float32),
    )(rows)


def _sc_query_group(ptx, pty, ptz, ctx, cty, ctz, ftf, B, N, K, C):
    mesh = plsc.VectorSubcoreMesh(core_axis_name="c", subcore_axis_name="s")
    NW = 32
    wpb = NW // B
    KW = K // wpb
    ncg = N // 64
    IDXW = 64
    nfc = KW * _NN // IDXW

    @functools.partial(
        pl.kernel,
        mesh=mesh,
        compiler_params=pltpu.CompilerParams(needs_layout_passes=False,
                                             use_tc_tiling_on_sc=False),
        out_type=[
            jax.ShapeDtypeStruct((B, 3, K, _NN), jnp.float32),
            jax.ShapeDtypeStruct((B * K * _NN, C), jnp.float32),
        ],
        scratch_types=[
            pltpu.VMEM((N,), jnp.float32),
            pltpu.VMEM((N,), jnp.float32),
            pltpu.VMEM((N,), jnp.float32),
            pltpu.VMEM((KW,), jnp.float32),
            pltpu.VMEM((KW,), jnp.float32),
            pltpu.VMEM((KW,), jnp.float32),
            pltpu.VMEM((N,), jnp.float32),
            pltpu.VMEM((N,), jnp.float32),
            pltpu.VMEM((N,), jnp.float32),
            pltpu.VMEM((128,), jnp.int32),
            pltpu.VMEM((nfc, IDXW), jnp.int32),
            pltpu.VMEM((KW, _NN), jnp.float32),
            pltpu.VMEM((KW, _NN), jnp.float32),
            pltpu.VMEM((KW, _NN), jnp.float32),
            pltpu.VMEM((2, IDXW, C), jnp.float32),
            pltpu.SemaphoreType.DMA,
            pltpu.SemaphoreType.DMA,
        ],
    )
    def sc_kern(ptx_h, pty_h, ptz_h, ctx_h, cty_h, ctz_h, ftf_h,
                gxyz_h, grows_h,
                xv, yv, zv, cxv, cyv, czv, xbv, ybv, p2v,
                scanv, gidxv, gxv, gyv, gzv,
                fbv, sem_g, sem_w):
        cid = lax.axis_index("c")
        sid = lax.axis_index("s")
        wid = sid * 2 + cid
        b = wid // wpb
        k0 = (wid % wpb) * KW

        pltpu.sync_copy(ptx_h.at[pl.ds(b * N, N)], xv)
        pltpu.sync_copy(pty_h.at[pl.ds(b * N, N)], yv)
        pltpu.sync_copy(ptz_h.at[pl.ds(b * N, N)], zv)
        pltpu.sync_copy(ctx_h.at[pl.ds(b * K + k0, KW)], cxv)
        pltpu.sync_copy(cty_h.at[pl.ds(b * K + k0, KW)], cyv)
        pltpu.sync_copy(ctz_h.at[pl.ds(b * K + k0, KW)], czv)

        lane = lax.iota(jnp.int32, 16)
        zero16 = jnp.zeros((16,), jnp.int32)
        row0 = (b * K + k0) * _NN

        def prep(n, carry):
            sl = pl.ds(n * 16, 16)
            x = xv[sl]
            y = yv[sl]
            z = zv[sl]
            xbv[sl] = _rne_bf16(x)
            ybv[sl] = _rne_bf16(y)
            p2v[sl] = (x * x + y * y) + z * z
            return carry

        lax.fori_loop(0, N // 16, prep, 0)

        def scan_centroid(i, j4):
            k = i * 2 + j4
            kf = zero16 + k
            cx = plsc.load_gather(cxv, [kf])
            cy = plsc.load_gather(cyv, [kf])
            cz = plsc.load_gather(czv, [kf])
            cxb = _rne_bf16(cx)
            cyb = _rne_bf16(cy)
            czb = _rne_bf16(cz)
            c2 = (cx * cx + cy * cy) + cz * cz

            def cond(st):
                g, cnt = st
                return (cnt < _NN) & (g < ncg)

            def body(st):
                g, cnt = st
                base = g * 64
                ms, sums = [], []
                for j in range(4):
                    sl = pl.ds(base + j * 16, 16)
                    cp = cxb * xbv[sl] + (cyb * ybv[sl]
                                          + czb * _rne_bf16(zv[sl]))
                    d2 = (c2 + p2v[sl]) - 2.0 * cp
                    m = d2 <= _R2
                    ms.append(m)
                    sums.append(jnp.sum(m.astype(jnp.int32)))
                c = cnt
                for j in range(4):
                    plsc.store_compressed(scanv.at[pl.ds(c, 16)],
                                          lane + (base + j * 16), mask=ms[j])
                    c = c + sums[j]
                return (g + 1, c)

            _, cnt = lax.while_loop(cond, body,
                                    (jnp.int32(0), jnp.int32(0)))

            cntv = zero16 + cnt
            for h in range(2):
                pos = lane + 16 * h
                posc = jnp.where(pos < cntv, pos, 0)
                vals = plsc.load_gather(scanv, [posc])
                vals = jnp.where(cntv > 0, vals, jnp.int32(N - 1))
                gxv[k, pl.ds(16 * h, 16)] = plsc.load_gather(xv, [vals]) - cx
                gyv[k, pl.ds(16 * h, 16)] = plsc.load_gather(yv, [vals]) - cy
                gzv[k, pl.ds(16 * h, 16)] = plsc.load_gather(zv, [vals]) - cz
                gidxv[i, pl.ds(j4 * _NN + 16 * h, 16)] = vals + b * N

        def per_group(i, carry):
            for j4 in range(2):
                scan_centroid(i, j4)

            @pl.when(i >= 2)
            def _():
                pltpu.make_async_copy(fbv.at[0],
                                      grows_h.at[pl.ds(row0, IDXW)],
                                      sem_w).wait()

            pltpu.async_copy(ftf_h.at[gidxv.at[i]], fbv.at[i % 2], sem_g)

            @pl.when(i >= 1)
            def _():
                pltpu.make_async_copy(ftf_h.at[gidxv.at[0]], fbv.at[0],
                                      sem_g).wait()
                pltpu.async_copy(fbv.at[(i + 1) % 2],
                                 grows_h.at[pl.ds(row0 + (i - 1) * IDXW,
                                                  IDXW)],
                                 sem_w)
            return carry

        lax.fori_loop(0, nfc, per_group, 0)

        pltpu.make_async_copy(ftf_h.at[gidxv.at[0]], fbv.at[0], sem_g).wait()
        pltpu.async_copy(fbv.at[(nfc - 1) % 2],
                         grows_h.at[pl.ds(row0 + (nfc - 1) * IDXW, IDXW)],
                         sem_w)
        pltpu.make_async_copy(fbv.at[0], grows_h.at[pl.ds(row0, IDXW)],
                              sem_w).wait()
        pltpu.make_async_copy(fbv.at[0], grows_h.at[pl.ds(row0, IDXW)],
                              sem_w).wait()

        pltpu.sync_copy(gxv, gxyz_h.at[b, 0, pl.ds(k0, KW), :])
        pltpu.sync_copy(gyv, gxyz_h.at[b, 1, pl.ds(k0, KW), :])
        pltpu.sync_copy(gzv, gxyz_h.at[b, 2, pl.ds(k0, KW), :])

    return sc_kern(ptx, pty, ptz, ctx, cty, ctz, ftf)


def kernel(points, centroids, features):
    B, N, _ = points.shape
    K = centroids.shape[1]
    C = features.shape[1]
    ptx = points[:, :, 0].reshape(B * N)
    pty = points[:, :, 1].reshape(B * N)
    ptz = points[:, :, 2].reshape(B * N)
    ctx = centroids[:, :, 0].reshape(B * K)
    cty = centroids[:, :, 1].reshape(B * K)
    ctz = centroids[:, :, 2].reshape(B * K)
    ftf = _transpose_in(features).reshape(B * N, C)
    gxyz, grows = _sc_query_group(ptx, pty, ptz, ctx, cty, ctz, ftf,
                                  B, N, K, C)
    gfeat = jnp.transpose(_transpose_out(grows, B, K), (0, 1, 3, 2))
    return (gxyz, gfeat)
